# baseline (speedup 1.0000x reference)
import functools

import jax
import jax.numpy as jnp
from jax import lax
from jax.experimental import pallas as pl
from jax.experimental.pallas import tpu as pltpu
from jax.experimental.pallas import tpu_sc as plsc

_LANES = 16
_CHUNK = 128
_LN2 = 0.6931471805599453


def _neg_log(s):
    bits = lax.bitcast_convert_type(s, jnp.int32)
    e = lax.shift_right_logical(bits, 23) - 127
    m = lax.bitcast_convert_type(
        (bits & 0x007FFFFF) | 0x3F800000, jnp.float32)
    big = m > 1.4142135623730951
    m = jnp.where(big, m * 0.5, m)
    ef = (e + jnp.where(big, 1, 0)).astype(jnp.float32)
    t = (m - 1.0) / (m + 1.0)
    t2 = t * t
    log_m = t * (2.0 + t2 * (0.6666666666666666
                             + t2 * (0.4 + t2 * 0.2857142857142857)))
    return -(ef * _LN2 + log_m)


def _sc_loss_partials(tbl16, ind_p, w_p, n_chunks, num_ch, hw, n_valid):
    rows_per_plane = hw // _LANES
    mesh = plsc.VectorSubcoreMesh(core_axis_name="c", subcore_axis_name="s")

    @functools.partial(
        pl.kernel,
        out_type=jax.ShapeDtypeStruct((2, _LANES), jnp.float32),
        mesh=mesh,
        scratch_types=[
            pltpu.VMEM((n_chunks, _CHUNK), jnp.int32),
            pltpu.VMEM((n_chunks, _CHUNK), jnp.float32),
            pltpu.VMEM((n_chunks, _CHUNK), jnp.int32),
            pltpu.VMEM((num_ch * n_chunks, _CHUNK), jnp.int32),
            pltpu.VMEM((n_chunks * _CHUNK, _LANES), jnp.float32),
            pltpu.VMEM((n_chunks * _CHUNK, _LANES), jnp.float32),
            pltpu.VMEM((n_chunks * _CHUNK, _LANES), jnp.float32),
            pltpu.VMEM((n_chunks * _CHUNK, _LANES), jnp.float32),
            pltpu.VMEM((2, _LANES), jnp.float32),
            pltpu.VMEM_SHARED((16, 2, _LANES), jnp.float32),
            pltpu.VMEM((16, 2, _LANES), jnp.float32),
            pltpu.VMEM((2, _LANES), jnp.float32),
            pltpu.SemaphoreType.DMA,
            [pltpu.SemaphoreType.DMA for _ in range(4)],
        ],
        compiler_params=pltpu.CompilerParams(
            needs_layout_passes=False, use_tc_tiling_on_sc=False),
    )
    def k(tbl_hbm, ind_hbm, w_hbm, out_hbm,
          ind_v, w_v, rem_v, idx_dma, g0, g1, g2, g3,
          acc_v, shared, all_v, res_v, sem0, sems):
        core = lax.axis_index("c")
        sid = lax.axis_index("s")
        b = sid * 2 + core
        cp1 = pltpu.async_copy(ind_hbm.at[b], ind_v, sem0)
        cp2 = pltpu.async_copy(w_hbm.at[b], w_v, sem0)
        cp1.wait()
        cp2.wait()
        gbufs = (g0, g1, g2, g3)
        for j in range(n_chunks):
            for t in range(_CHUNK // _LANES):
                sl = (j, pl.ds(t * _LANES, _LANES))
                iv = ind_v[sl]
                addr = (
                    lax.shift_right_logical(iv, 11) * 2048
                    + (lax.shift_right_logical(iv, 7) & 1) * 1024
                    + (lax.shift_right_logical(iv, 8) & 7) * 128
                    + (iv & 127))
                row = lax.shift_right_logical(addr, 4)
                rem_v[sl] = iv & (_LANES - 1)
                for c in range(num_ch):
                    base = (b * num_ch + c) * rows_per_plane
                    idx_dma[(c * n_chunks + j, pl.ds(t * _LANES, _LANES))] = (
                        row + base)
        for j in range(n_chunks):
            for c in range(num_ch):
                pltpu.async_copy(
                    tbl_hbm.at[idx_dma.at[c * n_chunks + j]],
                    gbufs[c].at[pl.ds(j * _CHUNK, _CHUNK)], sems[j])
        lane = lax.iota(jnp.int32, _LANES)
        acc_l = jnp.zeros((_LANES,), jnp.float32)
        acc_w = jnp.zeros((_LANES,), jnp.float32)
        for j in range(n_chunks):
            for c in range(num_ch):
                pltpu.make_async_copy(
                    tbl_hbm.at[idx_dma.at[c * n_chunks + j]],
                    gbufs[c].at[pl.ds(j * _CHUNK, _CHUNK)], sems[j]).wait()
            for t in range(_CHUNK // _LANES):
                sl = (j, pl.ds(t * _LANES, _LANES))
                rows16 = j * _CHUNK + t * _LANES + lane
                cols16 = rem_v[sl]
                w = w_v[sl]
                p0 = plsc.load_gather(g0, [rows16, cols16]) * w
                p1 = plsc.load_gather(g1, [rows16, cols16]) * w
                p2 = plsc.load_gather(g2, [rows16, cols16]) * w
                p3 = plsc.load_gather(g3, [rows16, cols16]) * w
                x1 = jnp.minimum(p0, p2) / (jnp.maximum(p0, p2) + 1e-6)
                x2 = jnp.minimum(p1, p3) / (jnp.maximum(p1, p3) + 1e-6)
                y = _neg_log(x1 + x2)
                pos = j * _CHUNK + t * _LANES
                if pos + _LANES > n_valid:
                    y = jnp.where(pos + lane < n_valid, y, 0.0)
                acc_l = acc_l + y
                acc_w = acc_w + w
        acc_v[0, :] = acc_l
        acc_v[1, :] = acc_w
        pltpu.sync_copy(acc_v, shared.at[sid])
        plsc.subcore_barrier()

        @pl.when(sid == 0)
        def _():
            pltpu.sync_copy(shared, all_v)
            suml = jnp.zeros((_LANES,), jnp.float32)
            sumw = jnp.zeros((_LANES,), jnp.float32)
            for i in range(16):
                suml = suml + all_v[i, 0, :]
                sumw = sumw + all_v[i, 1, :]
            ls = jnp.full((_LANES,), jnp.sum(suml))
            ws = jnp.full((_LANES,), jnp.sum(sumw))
            res_v[0, :] = jnp.where(lane == 0, ls, jnp.where(lane == 1, ws, 0.0))
            pltpu.sync_copy(res_v.at[0], out_hbm.at[core])

    return k(tbl16, ind_p, w_p)


def kernel(output, target, ind, weight):
    del target
    b, c, h, w = output.shape
    hw = h * w
    n = ind.shape[1]
    n_chunks = -(-n // _CHUNK)
    npad = n_chunks * _CHUNK

    tbl16 = (
        output.reshape(b, c, h // 8, 8, w // 128, 128)
        .transpose(0, 1, 2, 4, 3, 5)
        .reshape(b * c * hw // _LANES, _LANES))
    ind_p = jnp.zeros((b, npad), jnp.int32).at[:, :n].set(ind)
    w_p = jnp.zeros((b, npad), jnp.float32).at[:, :n].set(weight)

    parts = _sc_loss_partials(
        tbl16,
        ind_p.reshape(b, n_chunks, _CHUNK),
        w_p.reshape(b, n_chunks, _CHUNK),
        n_chunks, c, hw, n)

    return (parts[0, 0] + parts[1, 0]) / (parts[0, 1] + parts[1, 1] + 1e-6)


# speedup vs baseline: 1.0172x; 1.0172x over previous
import functools

import jax
import jax.numpy as jnp
from jax import lax
from jax.experimental import pallas as pl
from jax.experimental.pallas import tpu as pltpu
from jax.experimental.pallas import tpu_sc as plsc

_LANES = 16
_CHUNK = 128
_LN2 = 0.6931471805599453


def _neg_log(s):
    bits = lax.bitcast_convert_type(s, jnp.int32)
    e = lax.shift_right_logical(bits, 23) - 127
    m = lax.bitcast_convert_type(
        (bits & 0x007FFFFF) | 0x3F800000, jnp.float32)
    big = m > 1.4142135623730951
    m = jnp.where(big, m * 0.5, m)
    ef = (e + jnp.where(big, 1, 0)).astype(jnp.float32)
    t = (m - 1.0) / (m + 1.0)
    t2 = t * t
    log_m = t * (2.0 + t2 * (0.6666666666666666
                             + t2 * (0.4 + t2 * 0.2857142857142857)))
    return -(ef * _LN2 + log_m)


def _sc_loss_partials(tbl16, ind_p, w_p, n_chunks, num_ch, hw, n_valid):
    rows_per_plane = hw // _LANES
    mesh = plsc.VectorSubcoreMesh(core_axis_name="c", subcore_axis_name="s")

    @functools.partial(
        pl.kernel,
        out_type=jax.ShapeDtypeStruct((2, _LANES), jnp.float32),
        mesh=mesh,
        scratch_types=[
            pltpu.VMEM((n_chunks, _CHUNK), jnp.int32),
            pltpu.VMEM((n_chunks, _CHUNK), jnp.float32),
            pltpu.VMEM((n_chunks, _CHUNK), jnp.int32),
            pltpu.VMEM((num_ch * n_chunks, _CHUNK), jnp.int32),
            pltpu.VMEM((n_chunks * _CHUNK, _LANES), jnp.float32),
            pltpu.VMEM((n_chunks * _CHUNK, _LANES), jnp.float32),
            pltpu.VMEM((n_chunks * _CHUNK, _LANES), jnp.float32),
            pltpu.VMEM((n_chunks * _CHUNK, _LANES), jnp.float32),
            pltpu.VMEM((2, _LANES), jnp.float32),
            pltpu.VMEM_SHARED((16, 2, _LANES), jnp.float32),
            pltpu.VMEM((16, 2, _LANES), jnp.float32),
            pltpu.VMEM((2, _LANES), jnp.float32),
            pltpu.SemaphoreType.DMA,
            [pltpu.SemaphoreType.DMA for _ in range(4)],
        ],
        compiler_params=pltpu.CompilerParams(
            needs_layout_passes=False, use_tc_tiling_on_sc=False),
    )
    def k(tbl_hbm, ind_hbm, w_hbm, out_hbm,
          ind_v, w_v, rem_v, idx_dma, g0, g1, g2, g3,
          acc_v, shared, all_v, res_v, sem0, sems):
        core = lax.axis_index("c")
        sid = lax.axis_index("s")
        b = sid * 2 + core
        cp1 = pltpu.async_copy(ind_hbm.at[b], ind_v, sem0)
        cp2 = pltpu.async_copy(w_hbm.at[b], w_v, sem0)
        cp1.wait()
        cp2.wait()
        gbufs = (g0, g1, g2, g3)
        lane = lax.iota(jnp.int32, _LANES)
        for j in range(n_chunks):
            def addr_body(t, _, j=j):
                sl = (j, pl.ds(t * _LANES, _LANES))
                iv = ind_v[sl]
                addr = (
                    lax.shift_right_logical(iv, 11) * 2048
                    + (lax.shift_right_logical(iv, 7) & 1) * 1024
                    + (lax.shift_right_logical(iv, 8) & 7) * 128
                    + (iv & 127))
                row = lax.shift_right_logical(addr, 4)
                rem_v[sl] = iv & (_LANES - 1)
                for c in range(num_ch):
                    base = (b * num_ch + c) * rows_per_plane
                    idx_dma[(c * n_chunks + j, pl.ds(t * _LANES, _LANES))] = (
                        row + base)
                return 0
            lax.fori_loop(0, _CHUNK // _LANES, addr_body, 0)
            for c in range(num_ch):
                pltpu.async_copy(
                    tbl_hbm.at[idx_dma.at[c * n_chunks + j]],
                    gbufs[c].at[pl.ds(j * _CHUNK, _CHUNK)], sems[j])
        acc_l = jnp.zeros((_LANES,), jnp.float32)
        acc_w = jnp.zeros((_LANES,), jnp.float32)
        for j in range(n_chunks):
            for c in range(num_ch):
                pltpu.make_async_copy(
                    tbl_hbm.at[idx_dma.at[c * n_chunks + j]],
                    gbufs[c].at[pl.ds(j * _CHUNK, _CHUNK)], sems[j]).wait()

            def comp_body(t, carry, j=j):
                a_l, a_w = carry
                sl = (j, pl.ds(t * _LANES, _LANES))
                rows16 = j * _CHUNK + t * _LANES + lane
                cols16 = rem_v[sl]
                w = w_v[sl]
                p0 = plsc.load_gather(g0, [rows16, cols16]) * w
                p1 = plsc.load_gather(g1, [rows16, cols16]) * w
                p2 = plsc.load_gather(g2, [rows16, cols16]) * w
                p3 = plsc.load_gather(g3, [rows16, cols16]) * w
                x1 = jnp.minimum(p0, p2) / (jnp.maximum(p0, p2) + 1e-6)
                x2 = jnp.minimum(p1, p3) / (jnp.maximum(p1, p3) + 1e-6)
                y = _neg_log(x1 + x2)
                pos = j * _CHUNK + t * _LANES
                y = jnp.where(pos + lane < n_valid, y, 0.0)
                return (a_l + y, a_w + w)

            acc_l, acc_w = lax.fori_loop(
                0, _CHUNK // _LANES, comp_body, (acc_l, acc_w))
        acc_v[0, :] = acc_l
        acc_v[1, :] = acc_w
        pltpu.sync_copy(acc_v, shared.at[sid])
        plsc.subcore_barrier()

        @pl.when(sid == 0)
        def _():
            pltpu.sync_copy(shared, all_v)
            suml = jnp.zeros((_LANES,), jnp.float32)
            sumw = jnp.zeros((_LANES,), jnp.float32)
            for i in range(16):
                suml = suml + all_v[i, 0, :]
                sumw = sumw + all_v[i, 1, :]
            ls = jnp.full((_LANES,), jnp.sum(suml))
            ws = jnp.full((_LANES,), jnp.sum(sumw))
            res_v[0, :] = jnp.where(lane == 0, ls, jnp.where(lane == 1, ws, 0.0))
            pltpu.sync_copy(res_v.at[0], out_hbm.at[core])

    return k(tbl16, ind_p, w_p)


def kernel(output, target, ind, weight):
    del target
    b, c, h, w = output.shape
    hw = h * w
    n = ind.shape[1]
    n_chunks = -(-n // _CHUNK)
    npad = n_chunks * _CHUNK

    tbl16 = (
        output.reshape(b, c, h // 8, 8, w // 128, 128)
        .transpose(0, 1, 2, 4, 3, 5)
        .reshape(b * c * hw // _LANES, _LANES))
    ind_p = jnp.zeros((b, npad), jnp.int32).at[:, :n].set(ind)
    w_p = jnp.zeros((b, npad), jnp.float32).at[:, :n].set(weight)

    parts = _sc_loss_partials(
        tbl16,
        ind_p.reshape(b, n_chunks, _CHUNK),
        w_p.reshape(b, n_chunks, _CHUNK),
        n_chunks, c, hw, n)

    return (parts[0, 0] + parts[1, 0]) / (parts[0, 1] + parts[1, 1] + 1e-6)
